# baseline (speedup 1.0000x reference)
import functools

import jax
import jax.numpy as jnp
from jax import lax
from jax.experimental import pallas as pl
from jax.experimental.pallas import tpu as pltpu
from jax.experimental.pallas import tpu_sc as plsc

N_PIX = 262144
N_FEAT = 96
N_PAIRS = 32768
NC, NS, L = 2, 16, 16
NW = NC * NS
CHUNK_PIX = N_PIX // NW
POS_PER_W = N_PAIRS // NW
CCHUNK = 128
NCHUNK = POS_PER_W // CCHUNK
NBITS = 6 * N_PAIRS
GROUPS = ((0, 1, 1, 2), (2, 3, 2, 1), (4, 5, 0, 0))

_sc_mesh = plsc.VectorSubcoreMesh(core_axis_name="c", subcore_axis_name="s")


def _wid():
    return lax.axis_index("s") * NC + lax.axis_index("c")


def _mask_lists_body(mask_hbm, lists_hbm, counts_hbm, mbuf, lbuf, cbuf):
    w = _wid()
    pltpu.sync_copy(mask_hbm.at[pl.ds(w * CHUNK_PIX, CHUNK_PIX)], mbuf)
    base = w * CHUNK_PIX
    lane = lax.broadcasted_iota(jnp.int32, (L,), 0)
    offs = []
    for c in range(3):
        def step(j, off, c=c):
            mv = mbuf[pl.ds(j * L, L)]
            msk = mv == c
            mi = msk.astype(jnp.int32)
            pos = off + (plsc.cumsum(mi) - mi)
            pix = base + j * L + lane
            plsc.store_scatter(lbuf, [c * CHUNK_PIX + pos], pix, mask=msk)
            return off + jnp.sum(mi)
        off = lax.fori_loop(0, CHUNK_PIX // L, step, jnp.int32(0))
        offs.append(off)
        pltpu.sync_copy(lbuf.at[pl.ds(c * CHUNK_PIX, CHUNK_PIX)],
                        lists_hbm.at[pl.ds((c * NW + w) * CHUNK_PIX,
                                           CHUNK_PIX)])
    cvec = jnp.zeros((L,), jnp.int32)
    for c in range(3):
        cvec = jnp.where(lane == c, offs[c], cvec)
    cbuf[...] = cvec
    pltpu.sync_copy(cbuf.at[pl.ds(0, 8)],
                    counts_hbm.at[pl.ds(w * 8, 8)])


@functools.partial(
    pl.kernel,
    out_type=(jax.ShapeDtypeStruct((3 * N_PIX,), jnp.int32),
              jax.ShapeDtypeStruct((NW * 8,), jnp.int32)),
    mesh=_sc_mesh,
    compiler_params=pltpu.CompilerParams(needs_layout_passes=False),
    scratch_types=[
        pltpu.VMEM((CHUNK_PIX,), jnp.int32),
        pltpu.VMEM((3 * CHUNK_PIX,), jnp.int32),
        pltpu.VMEM((L,), jnp.int32),
    ],
)
def _mask_lists(mask_hbm, lists_hbm, counts_hbm, mbuf, lbuf, cbuf):
    _mask_lists_body(mask_hbm, lists_hbm, counts_hbm, mbuf, lbuf, cbuf)


FPAD = 128
TBLK = 16384


def _rng_body(out_ref):
    pltpu.prng_seed(42)
    bits = pltpu.prng_random_bits((NBITS // 128, 128))
    out_ref[...] = lax.bitcast_convert_type(bits, jnp.int32)


def _rng_bits():
    return pl.pallas_call(
        _rng_body,
        out_shape=jax.ShapeDtypeStruct((NBITS // 128, 128), jnp.int32),
    )()


def _tp_body(in_ref, out_ref):
    out_ref[:, 0:N_FEAT] = in_ref[...].T


def _transpose(features):
    return pl.pallas_call(
        _tp_body,
        grid=(N_PIX // TBLK,),
        in_specs=[pl.BlockSpec((N_FEAT, TBLK), lambda j: (0, j))],
        out_specs=pl.BlockSpec((TBLK, FPAD), lambda j: (j, 0)),
        out_shape=jax.ShapeDtypeStruct((N_PIX, FPAD), jnp.float32),
    )(features)


NQ = 2 * 3 * NCHUNK


def _sample_ids_body(bits_hbm, lists_hbm, counts_hbm, pix_hbm,
                     cntbuf, inclbuf, ebuf, bitsbuf, ibuf2, pixbuf, psem):
    w = _wid()
    pltpu.sync_copy(counts_hbm, cntbuf)
    for d in range(6):
        pltpu.sync_copy(
            bits_hbm.at[pl.ds(d * N_PAIRS + w * POS_PER_W, POS_PER_W)],
            bitsbuf.at[pl.ds(d * POS_PER_W, POS_PER_W)])
    lane = lax.broadcasted_iota(jnp.int32, (L,), 0)
    K = []
    for c in range(3):
        cnt_lo = plsc.load_gather(cntbuf, [lane * 8 + c])
        cnt_hi = plsc.load_gather(cntbuf, [(L + lane) * 8 + c])
        tot_lo = jnp.sum(cnt_lo)
        incl_lo = plsc.cumsum(cnt_lo)
        incl_hi = plsc.cumsum(cnt_hi) + tot_lo
        inclbuf[pl.ds(c * NW, L)] = incl_lo
        inclbuf[pl.ds(c * NW + L, L)] = incl_hi
        ebuf[pl.ds(c * NW, L)] = incl_lo - cnt_lo
        ebuf[pl.ds(c * NW + L, L)] = incl_hi - cnt_hi
        K.append(jnp.maximum(tot_lo + jnp.sum(cnt_hi), 1))

    for g, (dA, dB, cA, cB) in enumerate(GROUPS):
        for side, (d, c) in enumerate(((dA, cA), (dB, cB))):
            @plsc.parallel_loop(0, POS_PER_W // L, 1, unroll=2)
            def ijstep(jj, g=g, side=side, d=d, c=c):
                r = bitsbuf[pl.ds(d * POS_PER_W + jj * L, L)]
                r = lax.rem(r & jnp.int32(0x7FFFFFFF), K[c])
                v = jnp.zeros((L,), jnp.int32)
                for s in (16, 8, 4, 2, 1):
                    cand = v + s
                    val = plsc.load_gather(
                        inclbuf, [jnp.int32(c * NW) + cand - 1])
                    v = jnp.where(val <= r, cand, v)
                ev = plsc.load_gather(ebuf, [jnp.int32(c * NW) + v])
                q = (g * NCHUNK + jj // (CCHUNK // L)) * 2 + side
                ibuf2[q, pl.ds((jj % (CCHUNK // L)) * L, L)] = (
                    c * N_PIX + v * CHUNK_PIX + (r - ev))

    for q in range(NQ):
        pltpu.make_async_copy(
            lists_hbm.at[ibuf2.at[q]], pixbuf.at[q], psem).start()
    for q in range(NQ):
        pltpu.make_async_copy(
            lists_hbm.at[ibuf2.at[q]], pixbuf.at[q], psem).wait()
    pltpu.sync_copy(pixbuf, pix_hbm.at[w])


@functools.partial(
    pl.kernel,
    out_type=jax.ShapeDtypeStruct((NW, NQ, CCHUNK), jnp.int32),
    mesh=_sc_mesh,
    compiler_params=pltpu.CompilerParams(needs_layout_passes=False),
    scratch_types=[
        pltpu.VMEM((NW * 8,), jnp.int32),
        pltpu.VMEM((3 * NW,), jnp.int32),
        pltpu.VMEM((3 * NW,), jnp.int32),
        pltpu.VMEM((6 * POS_PER_W,), jnp.int32),
        pltpu.VMEM((NQ, CCHUNK), jnp.int32),
        pltpu.VMEM((NQ, CCHUNK), jnp.int32),
        pltpu.SemaphoreType.DMA,
    ],
)
def _sample_ids(bits_hbm, lists_hbm, counts_hbm, pix_hbm, *scratch):
    _sample_ids_body(bits_hbm, lists_hbm, counts_hbm, pix_hbm, *scratch)


def _gather_loss_body(pix_hbm, ft_hbm, out_hbm,
                      pixbuf, rA0, rA1, rA2, rB0, rB1, rB2, obuf,
                      sA0, sA1, sA2, sB0, sB1, sB2):
    w = _wid()
    pltpu.sync_copy(pix_hbm.at[w], pixbuf)

    rbufs = ((rA0, rB0, sA0, sB0), (rA1, rB1, sA1, sB1),
             (rA2, rB2, sA2, sB2))

    def fire(g24):
        rA, rB, sA, sB = rbufs[g24 % 3]
        pltpu.make_async_copy(
            ft_hbm.at[pixbuf.at[2 * g24]], rA, sA).start()
        pltpu.make_async_copy(
            ft_hbm.at[pixbuf.at[2 * g24 + 1]], rB, sB).start()

    def wait(g24):
        rA, rB, sA, sB = rbufs[g24 % 3]
        pltpu.make_async_copy(
            ft_hbm.at[pixbuf.at[2 * g24]], rA, sA).wait()
        pltpu.make_async_copy(
            ft_hbm.at[pixbuf.at[2 * g24 + 1]], rB, sB).wait()

    acc = jnp.zeros((L,), jnp.float32)
    fire(0)
    fire(1)
    for g24 in range(3 * NCHUNK):
        if g24 + 2 < 3 * NCHUNK:
            fire(g24 + 2)
        wait(g24)
        rA, rB, _, _ = rbufs[g24 % 3]

        @plsc.parallel_loop(0, CCHUNK, 1, unroll=4, carry=acc)
        def pstep(p, a, rA=rA, rB=rB):
            for t in range(N_FEAT // L):
                av = rA[p, pl.ds(t * L, L)]
                bv = rB[p, pl.ds(t * L, L)]
                dv = av - bv
                ad = jnp.abs(dv)
                a = a + jnp.where(ad < 1.0, 0.5 * dv * dv, ad - 0.5)
            return a
        acc = pstep
    obuf[...] = acc
    pltpu.sync_copy(obuf, out_hbm.at[pl.ds(w * L, L)])


@functools.partial(
    pl.kernel,
    out_type=jax.ShapeDtypeStruct((NW * L,), jnp.float32),
    mesh=_sc_mesh,
    compiler_params=pltpu.CompilerParams(needs_layout_passes=False),
    scratch_types=[
        pltpu.VMEM((NQ, CCHUNK), jnp.int32),
        pltpu.VMEM((CCHUNK, FPAD), jnp.float32),
        pltpu.VMEM((CCHUNK, FPAD), jnp.float32),
        pltpu.VMEM((CCHUNK, FPAD), jnp.float32),
        pltpu.VMEM((CCHUNK, FPAD), jnp.float32),
        pltpu.VMEM((CCHUNK, FPAD), jnp.float32),
        pltpu.VMEM((CCHUNK, FPAD), jnp.float32),
        pltpu.VMEM((L,), jnp.float32),
        pltpu.SemaphoreType.DMA,
        pltpu.SemaphoreType.DMA,
        pltpu.SemaphoreType.DMA,
        pltpu.SemaphoreType.DMA,
        pltpu.SemaphoreType.DMA,
        pltpu.SemaphoreType.DMA,
    ],
)
def _gather_loss(pix_hbm, ft_hbm, out_hbm, *scratch):
    _gather_loss_body(pix_hbm, ft_hbm, out_hbm, *scratch)


def kernel(features_flat, mask_flat):
    mask32 = mask_flat.astype(jnp.int32)
    ft = _transpose(features_flat)
    bits = _rng_bits().reshape(NBITS)
    lists, counts = _mask_lists(mask32)
    pix = _sample_ids(bits, lists, counts)
    partials = _gather_loss(pix, ft)
    return jnp.sum(partials) * (1.0 / (N_FEAT * N_PAIRS))


# speedup vs baseline: 1.1030x; 1.1030x over previous
import functools

import jax
import jax.numpy as jnp
from jax import lax
from jax.experimental import pallas as pl
from jax.experimental.pallas import tpu as pltpu
from jax.experimental.pallas import tpu_sc as plsc

N_PIX = 262144
N_FEAT = 96
N_PAIRS = 32768
NC, NS, L = 2, 16, 16
NW = NC * NS
CHUNK_PIX = N_PIX // NW
POS_PER_W = N_PAIRS // NW
CCHUNK = 128
NCHUNK = POS_PER_W // CCHUNK
NBITS = 6 * N_PAIRS
GROUPS = ((0, 1, 1, 2), (2, 3, 2, 1), (4, 5, 0, 0))

_sc_mesh = plsc.VectorSubcoreMesh(core_axis_name="c", subcore_axis_name="s")


def _wid():
    return lax.axis_index("s") * NC + lax.axis_index("c")


def _mask_lists_body(mask_hbm, lists_hbm, counts_hbm, mbuf, lbuf, cbuf):
    w = _wid()
    pltpu.sync_copy(mask_hbm.at[pl.ds(w * CHUNK_PIX, CHUNK_PIX)], mbuf)
    base = w * CHUNK_PIX
    lane = lax.broadcasted_iota(jnp.int32, (L,), 0)
    offs = []
    for c in range(3):
        def step(j, off, c=c):
            mv = mbuf[pl.ds(j * L, L)]
            msk = mv == c
            mi = msk.astype(jnp.int32)
            pos = off + (plsc.cumsum(mi) - mi)
            pix = base + j * L + lane
            plsc.store_scatter(lbuf, [c * CHUNK_PIX + pos], pix, mask=msk)
            return off + jnp.sum(mi)
        off = lax.fori_loop(0, CHUNK_PIX // L, step, jnp.int32(0))
        offs.append(off)
        pltpu.sync_copy(lbuf.at[pl.ds(c * CHUNK_PIX, CHUNK_PIX)],
                        lists_hbm.at[pl.ds((c * NW + w) * CHUNK_PIX,
                                           CHUNK_PIX)])
    cvec = jnp.zeros((L,), jnp.int32)
    for c in range(3):
        cvec = jnp.where(lane == c, offs[c], cvec)
    cbuf[...] = cvec
    pltpu.sync_copy(cbuf.at[pl.ds(0, 8)],
                    counts_hbm.at[pl.ds(w * 8, 8)])


@functools.partial(
    pl.kernel,
    out_type=(jax.ShapeDtypeStruct((3 * N_PIX,), jnp.int32),
              jax.ShapeDtypeStruct((NW * 8,), jnp.int32)),
    mesh=_sc_mesh,
    compiler_params=pltpu.CompilerParams(needs_layout_passes=False),
    scratch_types=[
        pltpu.VMEM((CHUNK_PIX,), jnp.int32),
        pltpu.VMEM((3 * CHUNK_PIX,), jnp.int32),
        pltpu.VMEM((L,), jnp.int32),
    ],
)
def _mask_lists(mask_hbm, lists_hbm, counts_hbm, mbuf, lbuf, cbuf):
    _mask_lists_body(mask_hbm, lists_hbm, counts_hbm, mbuf, lbuf, cbuf)


FPAD = 128
TBLK = 16384
BROWS = NBITS // 128 // (N_PIX // TBLK)


def _tp_body(in_ref, out_ref, bits_ref):
    out_ref[:, 0:N_FEAT] = in_ref[...].T
    pltpu.prng_seed(42 + pl.program_id(0))
    bits_ref[...] = lax.bitcast_convert_type(
        pltpu.prng_random_bits((1, BROWS, 128)), jnp.int32)


def _transpose_and_rng(features):
    return pl.pallas_call(
        _tp_body,
        grid=(N_PIX // TBLK,),
        in_specs=[pl.BlockSpec((N_FEAT, TBLK), lambda j: (0, j))],
        out_specs=[pl.BlockSpec((TBLK, FPAD), lambda j: (j, 0)),
                   pl.BlockSpec((1, BROWS, 128), lambda j: (j, 0, 0))],
        out_shape=[jax.ShapeDtypeStruct((N_PIX, FPAD), jnp.float32),
                   jax.ShapeDtypeStruct(
                       (N_PIX // TBLK, BROWS, 128), jnp.int32)],
    )(features)


NQ = 2 * 3 * NCHUNK


def _pair_loss_body(bits_hbm, lists_hbm, counts_hbm, ft_hbm, out_hbm,
                    cntbuf, inclbuf, ebuf, bitsbuf, ibuf2, pixbuf,
                    rA0, rA1, rA2, rB0, rB1, rB2, obuf,
                    psem, sA0, sA1, sA2, sB0, sB1, sB2):
    w = _wid()
    pltpu.sync_copy(counts_hbm, cntbuf)
    for d in range(6):
        pltpu.sync_copy(
            bits_hbm.at[pl.ds(d * N_PAIRS + w * POS_PER_W, POS_PER_W)],
            bitsbuf.at[pl.ds(d * POS_PER_W, POS_PER_W)])
    lane = lax.broadcasted_iota(jnp.int32, (L,), 0)
    K = []
    for c in range(3):
        cnt_lo = plsc.load_gather(cntbuf, [lane * 8 + c])
        cnt_hi = plsc.load_gather(cntbuf, [(L + lane) * 8 + c])
        tot_lo = jnp.sum(cnt_lo)
        incl_lo = plsc.cumsum(cnt_lo)
        incl_hi = plsc.cumsum(cnt_hi) + tot_lo
        inclbuf[pl.ds(c * NW, L)] = incl_lo
        inclbuf[pl.ds(c * NW + L, L)] = incl_hi
        ebuf[pl.ds(c * NW, L)] = incl_lo - cnt_lo
        ebuf[pl.ds(c * NW + L, L)] = incl_hi - cnt_hi
        K.append(jnp.maximum(tot_lo + jnp.sum(cnt_hi), 1))
    Kf = [K[c].astype(jnp.float32) for c in range(3)]

    for g, (dA, dB, cA, cB) in enumerate(GROUPS):
        for side, (d, c) in enumerate(((dA, cA), (dB, cB))):
            @plsc.parallel_loop(0, POS_PER_W // L, 1, unroll=2)
            def ijstep(jj, g=g, side=side, d=d, c=c):
                b = bitsbuf[pl.ds(d * POS_PER_W + jj * L, L)]
                u01 = lax.shift_right_logical(b, 8).astype(
                    jnp.float32) * jnp.float32(1.0 / 16777216.0)
                r = jnp.minimum((u01 * Kf[c]).astype(jnp.int32), K[c] - 1)
                v = jnp.zeros((L,), jnp.int32)
                for s in (16, 8, 4, 2, 1):
                    cand = v + s
                    val = plsc.load_gather(
                        inclbuf, [jnp.int32(c * NW) + cand - 1])
                    v = jnp.where(val <= r, cand, v)
                ev = plsc.load_gather(ebuf, [jnp.int32(c * NW) + v])
                q = (g * NCHUNK + jj // (CCHUNK // L)) * 2 + side
                ibuf2[q, pl.ds((jj % (CCHUNK // L)) * L, L)] = (
                    c * N_PIX + v * CHUNK_PIX + (r - ev))

    for q in range(NQ):
        pltpu.make_async_copy(
            lists_hbm.at[ibuf2.at[q]], pixbuf.at[q], psem).start()
    for q in range(NQ):
        pltpu.make_async_copy(
            lists_hbm.at[ibuf2.at[q]], pixbuf.at[q], psem).wait()

    rbufs = ((rA0, rB0, sA0, sB0), (rA1, rB1, sA1, sB1),
             (rA2, rB2, sA2, sB2))

    def fire(g24):
        rA, rB, sA, sB = rbufs[g24 % 3]
        pltpu.make_async_copy(ft_hbm.at[pixbuf.at[2 * g24]], rA, sA).start()
        pltpu.make_async_copy(
            ft_hbm.at[pixbuf.at[2 * g24 + 1]], rB, sB).start()

    def wait(g24):
        rA, rB, sA, sB = rbufs[g24 % 3]
        pltpu.make_async_copy(ft_hbm.at[pixbuf.at[2 * g24]], rA, sA).wait()
        pltpu.make_async_copy(
            ft_hbm.at[pixbuf.at[2 * g24 + 1]], rB, sB).wait()

    acc = jnp.zeros((L,), jnp.float32)
    fire(0)
    fire(1)
    for g24 in range(3 * NCHUNK):
        if g24 + 2 < 3 * NCHUNK:
            fire(g24 + 2)
        wait(g24)
        rA, rB, _, _ = rbufs[g24 % 3]

        @plsc.parallel_loop(0, CCHUNK, 1, unroll=4, carry=acc)
        def pstep(p, a, rA=rA, rB=rB):
            for t in range(N_FEAT // L):
                av = rA[p, pl.ds(t * L, L)]
                bv = rB[p, pl.ds(t * L, L)]
                dv = av - bv
                ad = jnp.abs(dv)
                a = a + jnp.where(ad < 1.0, 0.5 * dv * dv, ad - 0.5)
            return a
        acc = pstep
    obuf[...] = acc
    pltpu.sync_copy(obuf, out_hbm.at[pl.ds(w * L, L)])


@functools.partial(
    pl.kernel,
    out_type=jax.ShapeDtypeStruct((NW * L,), jnp.float32),
    mesh=_sc_mesh,
    compiler_params=pltpu.CompilerParams(needs_layout_passes=False),
    scratch_types=[
        pltpu.VMEM((NW * 8,), jnp.int32),
        pltpu.VMEM((3 * NW,), jnp.int32),
        pltpu.VMEM((3 * NW,), jnp.int32),
        pltpu.VMEM((6 * POS_PER_W,), jnp.int32),
        pltpu.VMEM((NQ, CCHUNK), jnp.int32),
        pltpu.VMEM((NQ, CCHUNK), jnp.int32),
        pltpu.VMEM((CCHUNK, FPAD), jnp.float32),
        pltpu.VMEM((CCHUNK, FPAD), jnp.float32),
        pltpu.VMEM((CCHUNK, FPAD), jnp.float32),
        pltpu.VMEM((CCHUNK, FPAD), jnp.float32),
        pltpu.VMEM((CCHUNK, FPAD), jnp.float32),
        pltpu.VMEM((CCHUNK, FPAD), jnp.float32),
        pltpu.VMEM((L,), jnp.float32),
        pltpu.SemaphoreType.DMA,
        pltpu.SemaphoreType.DMA,
        pltpu.SemaphoreType.DMA,
        pltpu.SemaphoreType.DMA,
        pltpu.SemaphoreType.DMA,
        pltpu.SemaphoreType.DMA,
        pltpu.SemaphoreType.DMA,
    ],
)
def _pair_loss(bits_hbm, lists_hbm, counts_hbm, ft_hbm, out_hbm, *scratch):
    _pair_loss_body(bits_hbm, lists_hbm, counts_hbm, ft_hbm, out_hbm, *scratch)


def kernel(features_flat, mask_flat):
    mask32 = mask_flat.astype(jnp.int32)
    ft, bits2d = _transpose_and_rng(features_flat)
    bits = bits2d.reshape(NBITS)
    lists, counts = _mask_lists(mask32)
    partials = _pair_loss(bits, lists, counts, ft)
    return jnp.sum(partials) * (1.0 / (N_FEAT * N_PAIRS))
